# baseline (speedup 1.0000x reference)
import functools

import jax
import jax.numpy as jnp
from jax import lax
from jax.experimental import pallas as pl
from jax.experimental.pallas import tpu as pltpu
from jax.experimental.pallas import tpu_sc as plsc

_NC = 2
_NS = 16
_L = 16
_CH = 256
_SUB = _CH // 128


def _v_table_body(rel_ref, w_ref, bias_ref, instr_ref, out_ref):
    p = lax.dot_general(rel_ref[...], w_ref[...],
                        (((1,), (1,)), ((), ())),
                        preferred_element_type=jnp.float32)
    p = p + bias_ref[...]
    H = p.shape[1]
    v = jnp.maximum(p * instr_ref[...].reshape(1, H), 0.0)
    out_ref[...] = v.astype(jnp.bfloat16)


def _build_v_table(rel_features, W, bias, instruction):
    NR, H = rel_features.shape
    B = instruction.shape[0]
    return pl.pallas_call(
        _v_table_body,
        grid=(B,),
        in_specs=[
            pl.BlockSpec((NR, H), lambda i: (0, 0)),
            pl.BlockSpec((H, H), lambda i: (0, 0)),
            pl.BlockSpec((1, H), lambda i: (0, 0)),
            pl.BlockSpec((1, 1, H), lambda i: (i, 0, 0)),
        ],
        out_specs=pl.BlockSpec((NR, H), lambda i: (i, 0)),
        out_shape=jax.ShapeDtypeStruct((B * NR, H), jnp.bfloat16),
    )(rel_features, W, bias.reshape(1, H), instruction.reshape(B, 1, H))


def _make_sc_scatter(BM, NF_pad, rows_acc, rows_per_tile, rows_sub, kchunks):
    per_tile = NF_pad // _NS
    n_chunks = per_tile // _CH
    curr_pad = ((BM + 48) // 16) * 16
    passes = kchunks // _NC

    mesh = plsc.VectorSubcoreMesh(core_axis_name="c", subcore_axis_name="s",
                                  num_cores=_NC, num_subcores=_NS)

    @functools.partial(
        pl.kernel,
        mesh=mesh,
        compiler_params=pltpu.CompilerParams(needs_layout_passes=False,
                                             use_tc_tiling_on_sc=False),
        out_type=jax.ShapeDtypeStruct((BM, kchunks, 32), jnp.float32),
        scratch_types=[
            pltpu.VMEM((3, _SUB, 128), jnp.int32),
            pltpu.VMEM((3, _SUB, 128), jnp.int32),
            pltpu.VMEM((_SUB, 128), jnp.int32),
            pltpu.VMEM((_SUB, 128), jnp.int32),
            pltpu.VMEM((_SUB, 128), jnp.int32),
            pltpu.VMEM((_SUB, 128), jnp.int32),
            pltpu.VMEM((_SUB, 128), jnp.int32),
            pltpu.VMEM((_SUB, 128), jnp.int32),
            pltpu.VMEM((_SUB, 128), jnp.float32),
            pltpu.VMEM((_SUB, 128), jnp.float32),
            pltpu.VMEM((_SUB, 128, 32), jnp.bfloat16),
            pltpu.VMEM((_SUB, 128, 32), jnp.bfloat16),
            pltpu.VMEM((_SUB, 128, 32), jnp.float32),
            pltpu.VMEM((rows_sub, 32), jnp.float32),
            pltpu.VMEM_SHARED((curr_pad,), jnp.float32),
            pltpu.VMEM_SHARED((rows_acc, 32), jnp.float32),
            pltpu.SemaphoreType.DMA, pltpu.SemaphoreType.DMA,
            pltpu.SemaphoreType.DMA, pltpu.SemaphoreType.DMA,
            pltpu.SemaphoreType.DMA, pltpu.SemaphoreType.DMA,
            pltpu.SemaphoreType.DMA,
        ],
    )
    def sc_kernel(v_hbm, curr_hbm, packed_hbm, out_hbm,
                  ib0, ib1, ix0, ix1, tb0, tb1, hb0, hb1, pb0, pb1,
                  rv0, rv1, rs, zbuf, curr_s, acc_s,
                  si0, si1, sv0, sv1, sp0, sp1, ssc):
        c = lax.axis_index("c")
        s = lax.axis_index("s")
        ib = (ib0, ib1)
        ix = (ix0, ix1)
        tb = (tb0, tb1)
        hb = (hb0, hb1)
        pb = (pb0, pb1)
        rv = (rv0, rv1)
        si = (si0, si1)
        sv = (sv0, sv1)
        sp = (sp0, sp1)

        @pl.when(s == 0)
        def _():
            pltpu.sync_copy(curr_hbm, curr_s)

        def zero_zbuf(r, _):
            z = jnp.zeros((_L,), jnp.float32)
            zbuf[r, 0:16] = z
            zbuf[r, 16:32] = z
            return 0
        lax.fori_loop(0, rows_sub, zero_zbuf, 0)
        plsc.subcore_barrier()

        for p in range(passes):
            k = c * passes + p

            def zero_acc(i, _):
                pltpu.sync_copy(
                    zbuf, acc_s.at[pl.ds(s * rows_per_tile + i * rows_sub,
                                         rows_sub), :])
                return 0
            lax.fori_loop(0, rows_per_tile // rows_sub, zero_acc, 0)
            plsc.subcore_barrier()

            ksplat = jnp.full((_L,), k, jnp.int32)
            cbase = s * n_chunks

            def drain_scatter(slot):
                for h in range(_SUB):
                    pltpu.make_async_copy(rs.at[h], acc_s.at[tb[slot].at[h]],
                                          ssc).wait()

            def prefetch(jp, slot, guard_tail):
                pltpu.make_async_copy(packed_hbm.at[cbase + jp], ib[slot],
                                      si[slot]).wait()
                @pl.when(jp >= 2)
                def _():
                    drain_scatter(slot)
                for h in range(_SUB):
                    for g in range(8):
                        sl = pl.ds(g * 16, 16)
                        ix[slot][h, sl] = ib[slot][0, h, sl] * 4 + ksplat
                        hb[slot][h, sl] = ib[slot][1, h, sl]
                        tb[slot][h, sl] = ib[slot][2, h, sl]
                if guard_tail:
                    @pl.when(jp + 2 < n_chunks)
                    def _():
                        pltpu.async_copy(packed_hbm.at[cbase + jp + 2],
                                         ib[slot], si[slot])
                else:
                    pltpu.async_copy(packed_hbm.at[cbase + jp + 2],
                                     ib[slot], si[slot])
                for h in range(_SUB):
                    pltpu.async_copy(v_hbm.at[ix[slot].at[h]],
                                     rv[slot].at[h], sv[slot])
                    pltpu.async_copy(curr_s.at[hb[slot].at[h]],
                                     pb[slot].at[h], sp[slot])

            def finish(j, slot):
                for h in range(_SUB):
                    pltpu.make_async_copy(curr_s.at[hb[slot].at[h]],
                                          pb[slot].at[h], sp[slot]).wait()
                    pltpu.make_async_copy(v_hbm.at[ix[slot].at[h]],
                                          rv[slot].at[h], sv[slot]).wait()
                for h in range(_SUB):
                    for g in range(8):
                        pv = pb[slot][h, pl.ds(g * 16, 16)]
                        for j16 in range(16):
                            f = g * 16 + j16
                            row = rv[slot][h, f, 0:32]
                            lo, hi = plsc.unpack(
                                row, format=plsc.PackFormat.INTERLEAVED)
                            spl = jnp.full((_L,), pv[j16], jnp.float32)
                            rs[h, f, 0:16] = lo * spl
                            rs[h, f, 16:32] = hi * spl
                for h in range(_SUB):
                    pltpu.async_copy(rs.at[h], acc_s.at[tb[slot].at[h]],
                                     ssc, add=True)

            pltpu.async_copy(packed_hbm.at[cbase], ib[0], si[0])
            pltpu.async_copy(packed_hbm.at[cbase + 1], ib[1], si[1])
            prefetch(jnp.int32(0), 0, False)

            def loop_body(jj, _):
                j = 2 * jj
                prefetch(j + 1, 1, True)
                finish(j, 0)

                @pl.when(j + 2 < n_chunks)
                def _():
                    prefetch(j + 2, 0, True)

                @pl.when(j + 2 >= n_chunks)
                def _():
                    drain_scatter(0)
                finish(j + 1, 1)
                return 0
            lax.fori_loop(0, n_chunks // 2, loop_body, 0)

            drain_scatter(1)
            plsc.subcore_barrier()

            last_start = (_NS - 1) * rows_per_tile
            last_rows = BM - last_start

            @pl.when(s < _NS - 1)
            def _():
                start = s * rows_per_tile
                pltpu.sync_copy(
                    acc_s.at[pl.ds(start, rows_per_tile), :],
                    out_hbm.at[pl.ds(start, rows_per_tile), k, :])

            @pl.when(s == _NS - 1)
            def _():
                pltpu.sync_copy(
                    acc_s.at[pl.ds(last_start, last_rows), :],
                    out_hbm.at[pl.ds(last_start, last_rows), k, :])
            plsc.subcore_barrier()

    return sc_kernel


def kernel(input_vector, curr_dist, instruction, rel_features, weight_list,
           W, b, batch_heads, batch_rels, batch_tails, batch_ids, fact_ids):
    B, M, H = input_vector.shape
    NR = rel_features.shape[0]
    NF = fact_ids.shape[0]
    BM = B * M
    kchunks = H // 32

    grp = jnp.arange(H) // 32
    pos = jnp.arange(H) % 32
    col_perm = grp * 32 + (pos % 2) * 16 + pos // 2
    W_p = W[col_perm, :]
    b_p = b[col_perm]
    instr_p = instruction[:, col_perm]

    per_tile = -(-NF // (_NS * 2 * _CH)) * (2 * _CH)
    NF_pad = per_tile * _NS
    pad = NF_pad - NF

    combo = (batch_ids.astype(jnp.int32) * NR + batch_rels.astype(jnp.int32))
    combo_p = jnp.concatenate([combo, jnp.zeros((pad,), jnp.int32)])
    heads_p = jnp.concatenate(
        [batch_heads.astype(jnp.int32), jnp.full((pad,), BM, jnp.int32)])
    tails_p = jnp.concatenate(
        [batch_tails.astype(jnp.int32), jnp.full((pad,), BM, jnp.int32)])
    packed = jnp.stack([combo_p, heads_p, tails_p]) \
        .reshape(3, NF_pad // _CH, _CH).transpose(1, 0, 2) \
        .reshape(NF_pad // _CH, 3, _SUB, 128)

    curr_pad = ((BM + 48) // 16) * 16
    curr_p = jnp.concatenate(
        [curr_dist.reshape(-1),
         jnp.zeros((curr_pad - BM,), jnp.float32)])

    rows_per_tile = -(-(BM + 32) // (_NS * 32)) * 32
    rows_acc = rows_per_tile * _NS
    rows_sub = rows_per_tile // 32

    v_tab = _build_v_table(rel_features, W_p, b_p, instr_p)
    v4 = v_tab.reshape(B * NR * 4, 32)

    sc = _make_sc_scatter(BM, NF_pad, rows_acc, rows_per_tile, rows_sub,
                          kchunks)
    out = sc(v4, curr_p, packed)
    return out.reshape(B, M, H)


# speedup vs baseline: 1.1564x; 1.1564x over previous
import functools

import jax
import jax.numpy as jnp
from jax import lax
from jax.experimental import pallas as pl
from jax.experimental.pallas import tpu as pltpu
from jax.experimental.pallas import tpu_sc as plsc

_NC = 2
_NS = 16
_L = 16
_CH = 128
_NB = 4


def _v_table_body(rel_ref, w_ref, bias_ref, instr_ref, out_ref):
    p = lax.dot_general(rel_ref[...], w_ref[...],
                        (((1,), (1,)), ((), ())),
                        preferred_element_type=jnp.float32)
    p = p + bias_ref[...]
    H = p.shape[1]
    out_ref[...] = jnp.maximum(p * instr_ref[...].reshape(1, H), 0.0)


def _build_v_table(rel_features, W, bias, instruction):
    NR, H = rel_features.shape
    B = instruction.shape[0]
    return pl.pallas_call(
        _v_table_body,
        grid=(B,),
        in_specs=[
            pl.BlockSpec((NR, H), lambda i: (0, 0)),
            pl.BlockSpec((H, H), lambda i: (0, 0)),
            pl.BlockSpec((1, H), lambda i: (0, 0)),
            pl.BlockSpec((1, 1, H), lambda i: (i, 0, 0)),
        ],
        out_specs=pl.BlockSpec((NR, H), lambda i: (i, 0)),
        out_shape=jax.ShapeDtypeStruct((B * NR, H), jnp.float32),
    )(rel_features, W, bias.reshape(1, H), instruction.reshape(B, 1, H))


def _make_sc_scatter(BM, NF_pad, rows_acc, rows_per_tile, rows_sub, kchunks):
    per_tile = NF_pad // _NS
    n_chunks = per_tile // _CH
    curr_pad = ((BM + 48) // 16) * 16
    passes = kchunks // _NC

    mesh = plsc.VectorSubcoreMesh(core_axis_name="c", subcore_axis_name="s",
                                  num_cores=_NC, num_subcores=_NS)

    def _nb(t):
        return [t] * _NB

    @functools.partial(
        pl.kernel,
        mesh=mesh,
        compiler_params=pltpu.CompilerParams(needs_layout_passes=False,
                                             use_tc_tiling_on_sc=False),
        out_type=jax.ShapeDtypeStruct((BM, kchunks, 32), jnp.float32),
        scratch_types=(
            _nb(pltpu.VMEM((3, _CH), jnp.int32))
            + _nb(pltpu.VMEM((1, _CH), jnp.int32))
            + _nb(pltpu.VMEM((1, _CH), jnp.int32))
            + _nb(pltpu.VMEM((1, _CH), jnp.int32))
            + _nb(pltpu.VMEM((1, _CH), jnp.float32))
            + _nb(pltpu.VMEM((1, _CH, 32), jnp.float32))
            + [pltpu.VMEM((rows_sub, 32), jnp.float32),
               pltpu.VMEM_SHARED((curr_pad,), jnp.float32),
               pltpu.VMEM_SHARED((rows_acc, 32), jnp.float32)]
            + _nb(pltpu.SemaphoreType.DMA)
            + _nb(pltpu.SemaphoreType.DMA)
            + _nb(pltpu.SemaphoreType.DMA)
            + _nb(pltpu.SemaphoreType.DMA)
        ),
    )
    def sc_kernel(v_hbm, curr_hbm, packed_hbm, out_hbm, *bufs):
        ib = bufs[0:4]
        ix = bufs[4:8]
        tb = bufs[8:12]
        hb = bufs[12:16]
        pb = bufs[16:20]
        rv = bufs[20:24]
        zbuf = bufs[24]
        curr_s = bufs[25]
        acc_s = bufs[26]
        si = bufs[27:31]
        sv = bufs[31:35]
        sp = bufs[35:39]
        ss = bufs[39:43]
        c = lax.axis_index("c")
        s = lax.axis_index("s")

        @pl.when(s == 0)
        def _():
            pltpu.sync_copy(curr_hbm, curr_s)

        def zero_zbuf(r, _):
            z = jnp.zeros((_L,), jnp.float32)
            zbuf[r, 0:16] = z
            zbuf[r, 16:32] = z
            return 0
        lax.fori_loop(0, rows_sub, zero_zbuf, 0)
        plsc.subcore_barrier()

        for p in range(passes):
            k = c * passes + p

            def zero_acc(i, _):
                pltpu.sync_copy(
                    zbuf, acc_s.at[pl.ds(s * rows_per_tile + i * rows_sub,
                                         rows_sub), :])
                return 0
            lax.fori_loop(0, rows_per_tile // rows_sub, zero_acc, 0)
            plsc.subcore_barrier()

            ksplat = jnp.full((_L,), k, jnp.int32)
            cbase = s * n_chunks

            def drain_scatter(slot):
                pltpu.make_async_copy(rv[slot].at[0],
                                      acc_s.at[tb[slot].at[0]],
                                      ss[slot]).wait()

            def prefetch(jp, slot):
                pltpu.make_async_copy(packed_hbm.at[cbase + jp], ib[slot],
                                      si[slot]).wait()
                @pl.when(jp >= _NB)
                def _():
                    drain_scatter(slot)
                for g in range(8):
                    sl = pl.ds(g * 16, 16)
                    ix[slot][0, sl] = ib[slot][0, sl] * 4 + ksplat
                    hb[slot][0, sl] = ib[slot][1, sl]
                    tb[slot][0, sl] = ib[slot][2, sl]
                @pl.when(jp + _NB < n_chunks)
                def _():
                    pltpu.async_copy(packed_hbm.at[cbase + jp + _NB],
                                     ib[slot], si[slot])
                pltpu.async_copy(v_hbm.at[ix[slot].at[0]], rv[slot].at[0],
                                 sv[slot])
                pltpu.async_copy(curr_s.at[hb[slot].at[0]], pb[slot].at[0],
                                 sp[slot])

            def finish(j, slot):
                pltpu.make_async_copy(curr_s.at[hb[slot].at[0]],
                                      pb[slot].at[0], sp[slot]).wait()
                pltpu.make_async_copy(v_hbm.at[ix[slot].at[0]],
                                      rv[slot].at[0], sv[slot]).wait()
                for g in range(8):
                    pv = pb[slot][0, pl.ds(g * 16, 16)]
                    for j16 in range(16):
                        f = g * 16 + j16
                        spl = jnp.full((_L,), pv[j16], jnp.float32)
                        rv[slot][0, f, 0:16] = rv[slot][0, f, 0:16] * spl
                        rv[slot][0, f, 16:32] = rv[slot][0, f, 16:32] * spl
                pltpu.async_copy(rv[slot].at[0], acc_s.at[tb[slot].at[0]],
                                 ss[slot], add=True)

            for u in range(_NB):
                pltpu.async_copy(packed_hbm.at[cbase + u], ib[u], si[u])
            prefetch(jnp.int32(0), 0)
            prefetch(jnp.int32(1), 1)

            def loop_body(jj, _):
                j = _NB * jj
                for u in range(_NB):
                    jp = j + u + 2

                    @pl.when(jp < n_chunks)
                    def _():
                        prefetch(jp, (u + 2) % _NB)
                    finish(j + u, u)
                return 0
            lax.fori_loop(0, n_chunks // _NB, loop_body, 0)

            for u in range(_NB):
                drain_scatter(u)
            plsc.subcore_barrier()

            last_start = (_NS - 1) * rows_per_tile
            last_rows = BM - last_start

            @pl.when(s < _NS - 1)
            def _():
                start = s * rows_per_tile
                pltpu.sync_copy(
                    acc_s.at[pl.ds(start, rows_per_tile), :],
                    out_hbm.at[pl.ds(start, rows_per_tile), k, :])

            @pl.when(s == _NS - 1)
            def _():
                pltpu.sync_copy(
                    acc_s.at[pl.ds(last_start, last_rows), :],
                    out_hbm.at[pl.ds(last_start, last_rows), k, :])
            plsc.subcore_barrier()

    return sc_kernel


def kernel(input_vector, curr_dist, instruction, rel_features, weight_list,
           W, b, batch_heads, batch_rels, batch_tails, batch_ids, fact_ids):
    B, M, H = input_vector.shape
    NR = rel_features.shape[0]
    NF = fact_ids.shape[0]
    BM = B * M
    kchunks = H // 32

    per_tile = -(-NF // (_NS * _NB * _CH)) * (_NB * _CH)
    NF_pad = per_tile * _NS
    pad = NF_pad - NF

    combo = (batch_ids.astype(jnp.int32) * NR + batch_rels.astype(jnp.int32))
    combo_p = jnp.concatenate([combo, jnp.zeros((pad,), jnp.int32)])
    heads_p = jnp.concatenate(
        [batch_heads.astype(jnp.int32), jnp.full((pad,), BM, jnp.int32)])
    tails_p = jnp.concatenate(
        [batch_tails.astype(jnp.int32), jnp.full((pad,), BM, jnp.int32)])
    packed = jnp.stack([combo_p, heads_p, tails_p]) \
        .reshape(3, NF_pad // _CH, _CH).transpose(1, 0, 2)

    curr_pad = ((BM + 48) // 16) * 16
    curr_p = jnp.concatenate(
        [curr_dist.reshape(-1),
         jnp.zeros((curr_pad - BM,), jnp.float32)])

    rows_per_tile = -(-(BM + 32) // (_NS * 32)) * 32
    rows_acc = rows_per_tile * _NS
    rows_sub = rows_per_tile // 32

    v_tab = _build_v_table(rel_features, W, b, instruction)
    v4 = v_tab.reshape(B * NR * 4, 32)

    sc = _make_sc_scatter(BM, NF_pad, rows_acc, rows_per_tile, rows_sub,
                          kchunks)
    out = sc(v4, curr_p, packed)
    return out.reshape(B, M, H)


# speedup vs baseline: 1.1632x; 1.0059x over previous
import functools

import jax
import jax.numpy as jnp
from jax import lax
from jax.experimental import pallas as pl
from jax.experimental.pallas import tpu as pltpu
from jax.experimental.pallas import tpu_sc as plsc

_NC = 2
_NS = 16
_L = 16
_CH = 128
_NB = 4


def _v_table_body(rel_ref, w_ref, bias_ref, instr_ref, out_ref):
    p = lax.dot_general(rel_ref[...], w_ref[...],
                        (((1,), (1,)), ((), ())),
                        preferred_element_type=jnp.float32)
    p = p + bias_ref[...]
    H = p.shape[1]
    out_ref[...] = jnp.maximum(p * instr_ref[...].reshape(1, H), 0.0)


def _build_v_table(rel_features, W, bias, instruction):
    NR, H = rel_features.shape
    B = instruction.shape[0]
    return pl.pallas_call(
        _v_table_body,
        grid=(B,),
        in_specs=[
            pl.BlockSpec((NR, H), lambda i: (0, 0)),
            pl.BlockSpec((H, H), lambda i: (0, 0)),
            pl.BlockSpec((1, H), lambda i: (0, 0)),
            pl.BlockSpec((1, 1, H), lambda i: (i, 0, 0)),
        ],
        out_specs=pl.BlockSpec((NR, H), lambda i: (i, 0)),
        out_shape=jax.ShapeDtypeStruct((B * NR, H), jnp.float32),
    )(rel_features, W, bias.reshape(1, H), instruction.reshape(B, 1, H))


def _make_sc_scatter(BM, NF_pad, rows_acc, rows_per_tile, rows_sub, kchunks):
    per_tile = NF_pad // _NS
    n_chunks = per_tile // _CH
    curr_pad = ((BM + 48) // 16) * 16
    passes = kchunks // _NC

    mesh = plsc.VectorSubcoreMesh(core_axis_name="c", subcore_axis_name="s",
                                  num_cores=_NC, num_subcores=_NS)

    def _nb(t):
        return [t] * _NB

    @functools.partial(
        pl.kernel,
        mesh=mesh,
        compiler_params=pltpu.CompilerParams(needs_layout_passes=False,
                                             use_tc_tiling_on_sc=False),
        out_type=jax.ShapeDtypeStruct((BM, kchunks, 32), jnp.float32),
        scratch_types=(
            _nb(pltpu.VMEM((3, _CH), jnp.int32))
            + _nb(pltpu.VMEM((1, _CH), jnp.int32))
            + _nb(pltpu.VMEM((1, _CH), jnp.int32))
            + _nb(pltpu.VMEM((1, _CH), jnp.int32))
            + _nb(pltpu.VMEM((1, _CH), jnp.float32))
            + _nb(pltpu.VMEM((1, _CH, 32), jnp.float32))
            + [pltpu.VMEM((rows_sub, 32), jnp.float32),
               pltpu.VMEM_SHARED((curr_pad,), jnp.float32),
               pltpu.VMEM_SHARED((rows_acc, 32), jnp.float32)]
            + _nb(pltpu.SemaphoreType.DMA)
            + _nb(pltpu.SemaphoreType.DMA)
            + _nb(pltpu.SemaphoreType.DMA)
            + _nb(pltpu.SemaphoreType.DMA)
        ),
    )
    def sc_kernel(v_hbm, curr_hbm, packed_hbm, out_hbm, *bufs):
        ib = bufs[0:4]
        ix = bufs[4:8]
        tb = bufs[8:12]
        hb = bufs[12:16]
        pb = bufs[16:20]
        rv = bufs[20:24]
        zbuf = bufs[24]
        curr_s = bufs[25]
        acc_s = bufs[26]
        si = bufs[27:31]
        sv = bufs[31:35]
        sp = bufs[35:39]
        ss = bufs[39:43]
        c = lax.axis_index("c")
        s = lax.axis_index("s")

        @pl.when(s == 0)
        def _():
            pltpu.sync_copy(curr_hbm, curr_s)

        def zero_zbuf(r, _):
            z = jnp.zeros((_L,), jnp.float32)
            zbuf[r, 0:16] = z
            zbuf[r, 16:32] = z
            return 0
        lax.fori_loop(0, rows_sub, zero_zbuf, 0)
        plsc.subcore_barrier()

        for p in range(passes):
            k = c * passes + p
            ksplat = jnp.full((_L,), k, jnp.int32)
            cbase = s * n_chunks

            def drain_scatter(slot):
                pltpu.make_async_copy(rv[slot].at[0],
                                      acc_s.at[tb[slot].at[0]],
                                      ss[slot]).wait()

            def prefetch(jp, slot):
                pltpu.make_async_copy(packed_hbm.at[cbase + jp], ib[slot],
                                      si[slot]).wait()
                @pl.when(jp >= _NB)
                def _():
                    drain_scatter(slot)
                for g in range(8):
                    sl = pl.ds(g * 16, 16)
                    ix[slot][0, sl] = ib[slot][0, sl] * 4 + ksplat
                    hb[slot][0, sl] = ib[slot][1, sl]
                    tb[slot][0, sl] = ib[slot][2, sl]
                @pl.when(jp + _NB < n_chunks)
                def _():
                    pltpu.async_copy(packed_hbm.at[cbase + jp + _NB],
                                     ib[slot], si[slot])
                pltpu.async_copy(v_hbm.at[ix[slot].at[0]], rv[slot].at[0],
                                 sv[slot])
                pltpu.async_copy(curr_s.at[hb[slot].at[0]], pb[slot].at[0],
                                 sp[slot])

            def finish(j, slot):
                pltpu.make_async_copy(curr_s.at[hb[slot].at[0]],
                                      pb[slot].at[0], sp[slot]).wait()
                pltpu.make_async_copy(v_hbm.at[ix[slot].at[0]],
                                      rv[slot].at[0], sv[slot]).wait()
                for g in range(8):
                    pv = pb[slot][0, pl.ds(g * 16, 16)]
                    for j16 in range(16):
                        f = g * 16 + j16
                        spl = jnp.full((_L,), pv[j16], jnp.float32)
                        rv[slot][0, f, 0:16] = rv[slot][0, f, 0:16] * spl
                        rv[slot][0, f, 16:32] = rv[slot][0, f, 16:32] * spl
                pltpu.async_copy(rv[slot].at[0], acc_s.at[tb[slot].at[0]],
                                 ss[slot], add=True)

            for u in range(_NB):
                pltpu.async_copy(packed_hbm.at[cbase + u], ib[u], si[u])
            prefetch(jnp.int32(0), 0)
            prefetch(jnp.int32(1), 1)

            def zero_acc(i, _):
                pltpu.sync_copy(
                    zbuf, acc_s.at[pl.ds(s * rows_per_tile + i * rows_sub,
                                         rows_sub), :])
                return 0
            lax.fori_loop(0, rows_per_tile // rows_sub, zero_acc, 0)
            plsc.subcore_barrier()

            def loop_body(jj, _):
                j = _NB * jj
                for u in range(_NB):
                    jp = j + u + 2

                    @pl.when(jp < n_chunks)
                    def _():
                        prefetch(jp, (u + 2) % _NB)
                    finish(j + u, u)
                return 0
            lax.fori_loop(0, n_chunks // _NB, loop_body, 0)

            for u in range(_NB):
                drain_scatter(u)
            plsc.subcore_barrier()

            last_start = (_NS - 1) * rows_per_tile
            last_rows = BM - last_start

            @pl.when(s < _NS - 1)
            def _():
                start = s * rows_per_tile
                pltpu.sync_copy(
                    acc_s.at[pl.ds(start, rows_per_tile), :],
                    out_hbm.at[pl.ds(start, rows_per_tile), k, :])

            @pl.when(s == _NS - 1)
            def _():
                pltpu.sync_copy(
                    acc_s.at[pl.ds(last_start, last_rows), :],
                    out_hbm.at[pl.ds(last_start, last_rows), k, :])
            plsc.subcore_barrier()

    return sc_kernel


def kernel(input_vector, curr_dist, instruction, rel_features, weight_list,
           W, b, batch_heads, batch_rels, batch_tails, batch_ids, fact_ids):
    B, M, H = input_vector.shape
    NR = rel_features.shape[0]
    NF = fact_ids.shape[0]
    BM = B * M
    kchunks = H // 32

    per_tile = -(-NF // (_NS * _NB * _CH)) * (_NB * _CH)
    NF_pad = per_tile * _NS
    pad = NF_pad - NF

    combo = (batch_ids.astype(jnp.int32) * NR + batch_rels.astype(jnp.int32))
    combo_p = jnp.concatenate([combo, jnp.zeros((pad,), jnp.int32)])
    heads_p = jnp.concatenate(
        [batch_heads.astype(jnp.int32), jnp.full((pad,), BM, jnp.int32)])
    tails_p = jnp.concatenate(
        [batch_tails.astype(jnp.int32), jnp.full((pad,), BM, jnp.int32)])
    packed = jnp.stack([combo_p, heads_p, tails_p]) \
        .reshape(3, NF_pad // _CH, _CH).transpose(1, 0, 2)

    curr_pad = ((BM + 48) // 16) * 16
    curr_p = jnp.concatenate(
        [curr_dist.reshape(-1),
         jnp.zeros((curr_pad - BM,), jnp.float32)])

    rows_per_tile = -(-(BM + 32) // (_NS * 32)) * 32
    rows_acc = rows_per_tile * _NS
    rows_sub = rows_per_tile // 16

    v_tab = _build_v_table(rel_features, W, b, instruction)
    v4 = v_tab.reshape(B * NR * 4, 32)

    sc = _make_sc_scatter(BM, NF_pad, rows_acc, rows_per_tile, rows_sub,
                          kchunks)
    out = sc(v4, curr_p, packed)
    return out.reshape(B, M, H)
